# baseline (speedup 1.0000x reference)
import functools

import jax
import jax.numpy as jnp
from jax.experimental import pallas as pl
from jax.experimental.pallas import tpu as pltpu

N_NODES = 4096
FEATS = 256
TILE_M = 512
BUFS = 3


def _gcn_outer(x_ref, adj_hbm, w_ref, b_ref, out_hbm):
    def inner(adj_blk, out_blk):
        t = jnp.dot(adj_blk[...], x_ref[...],
                    preferred_element_type=jnp.float32,
                    precision=jax.lax.Precision.DEFAULT)
        out_blk[...] = jnp.dot(t, w_ref[...],
                               preferred_element_type=jnp.float32,
                               precision=jax.lax.Precision.DEFAULT) + b_ref[...]

    n = adj_hbm.shape[0]
    pipeline = pltpu.emit_pipeline(
        inner,
        grid=(n // TILE_M,),
        in_specs=[
            pl.BlockSpec((TILE_M, n), lambda i: (i, 0),
                         pipeline_mode=pl.Buffered(buffer_count=BUFS,
                                                   use_lookahead=False)),
        ],
        out_specs=[pl.BlockSpec((TILE_M, FEATS), lambda i: (i, 0))],
    )
    pipeline(adj_hbm, out_hbm)


@functools.partial(jax.jit, static_argnames=())
def kernel(input, adj, W, b):
    n, f_in = input.shape
    f_out = W.shape[1]
    b2 = b.reshape(1, f_out)
    return pl.pallas_call(
        _gcn_outer,
        in_specs=[
            pl.BlockSpec(memory_space=pltpu.MemorySpace.VMEM),
            pl.BlockSpec(memory_space=pltpu.MemorySpace.HBM),
            pl.BlockSpec(memory_space=pltpu.MemorySpace.VMEM),
            pl.BlockSpec(memory_space=pltpu.MemorySpace.VMEM),
        ],
        out_specs=pl.BlockSpec(memory_space=pltpu.MemorySpace.HBM),
        out_shape=jax.ShapeDtypeStruct((n, f_out), jnp.float32),
    )(input, adj, W, b2)


# speedup vs baseline: 1.0470x; 1.0470x over previous
import functools

import jax
import jax.numpy as jnp
from jax.experimental import pallas as pl
from jax.experimental.pallas import tpu as pltpu

N_NODES = 4096
FEATS = 256
TILE_M = 512


def _gcn_block(x_ref, adj_ref, w_ref, b_ref, out_ref):
    adj_bf = adj_ref[...].astype(jnp.bfloat16)
    x_bf = x_ref[...].astype(jnp.bfloat16)
    t = jnp.dot(adj_bf, x_bf, preferred_element_type=jnp.float32)
    w_bf = w_ref[...].astype(jnp.bfloat16)
    out = jnp.dot(t.astype(jnp.bfloat16), w_bf, preferred_element_type=jnp.float32)
    out_ref[...] = out + b_ref[...]


@functools.partial(jax.jit, static_argnames=())
def kernel(input, adj, W, b):
    n, f_in = input.shape
    f_out = W.shape[1]
    b2 = b.reshape(1, f_out)
    grid = (n // TILE_M,)
    return pl.pallas_call(
        _gcn_block,
        grid=grid,
        in_specs=[
            pl.BlockSpec((n, f_in), lambda i: (0, 0)),
            pl.BlockSpec((TILE_M, n), lambda i: (i, 0)),
            pl.BlockSpec((f_in, f_out), lambda i: (0, 0)),
            pl.BlockSpec((1, f_out), lambda i: (0, 0)),
        ],
        out_specs=pl.BlockSpec((TILE_M, f_out), lambda i: (i, 0)),
        out_shape=jax.ShapeDtypeStruct((n, f_out), jnp.float32),
        compiler_params=pltpu.CompilerParams(
            dimension_semantics=("parallel",),
        ),
    )(input, adj, W, b2)
